# baseline (speedup 1.0000x reference)
import functools

import jax
import jax.numpy as jnp
from jax import lax
from itertools import product as _prod
from jax.experimental import pallas as pl
from jax.experimental.pallas import tpu as pltpu
from jax.experimental.pallas import tpu_sc as plsc

_VSZ = 0.1
_FD = 128
_NBLK = 2
_NTILE = 2000

_UH_PAD = 65536
_BIG = jnp.iinfo(jnp.int32).max
_NGROW = 11264


def _sc_nbsearch(uh_pad, gidx2):
    mesh = plsc.VectorSubcoreMesh(core_axis_name="c", subcore_axis_name="s",
                                  num_cores=2)

    blk = _NGROW * 128 // 32 // 4

    @functools.partial(
        pl.kernel,
        out_type=jax.ShapeDtypeStruct((_NGROW * 128,), jnp.int32),
        mesh=mesh,
        compiler_params=pltpu.CompilerParams(needs_layout_passes=False),
        scratch_types=[
            pltpu.VMEM((_UH_PAD,), jnp.int32),
            pltpu.VMEM((blk,), jnp.int32),
            pltpu.VMEM((blk,), jnp.int32),
        ],
    )
    def k(uh_ref, gidx_ref, out_ref, uh_v, gix_v, res_v):
        w = lax.axis_index("s") * 2 + lax.axis_index("c")
        pltpu.sync_copy(uh_ref, uh_v)

        @pl.loop(0, 4)
        def _(b):
            base = (w * 4 + b) * blk
            pltpu.sync_copy(gidx_ref.at[pl.ds(base, blk)], gix_v)

            @pl.loop(0, blk // 128)
            def _(r):
                qs = [gix_v[pl.ds(r * 128 + j * 16, 16)] for j in range(8)]
                ps = [jnp.zeros((16,), jnp.int32) for _ in range(8)]
                s = _UH_PAD // 2
                while s >= 1:
                    probes = [plsc.load_gather(uh_v, [ps[j] + (s - 1)])
                              for j in range(8)]
                    ps = [jnp.where(probes[j] < qs[j], ps[j] + s, ps[j])
                          for j in range(8)]
                    s //= 2
                for j in range(8):
                    val = plsc.load_gather(uh_v, [ps[j]])
                    res_v[pl.ds(r * 128 + j * 16, 16)] = jnp.where(
                        val == qs[j], ps[j], -1)

            pltpu.sync_copy(res_v, out_ref.at[pl.ds(base, blk)])

    return k(uh_pad, gidx2)


def _structure(points):
    N = points.shape[0]
    coords = jnp.floor(points / _VSZ).astype(jnp.int32)
    coords = coords - coords.min(axis=0)
    mx = coords.max(axis=0) + 1
    mx1, mx2 = mx[1], mx[2]
    h = coords[:, 0] * (mx1 * mx2) + coords[:, 1] * mx2 + coords[:, 2]
    uh, inv = jnp.unique(h, return_inverse=True, size=N, fill_value=-1)
    inv = inv.reshape(-1).astype(jnp.int32)
    M = jnp.sum(uh >= 0).astype(jnp.int32)
    row_valid = jnp.arange(N, dtype=jnp.int32) < M
    c0 = uh // (mx1 * mx2)
    r = uh % (mx1 * mx2)
    c1 = r // mx2
    c2 = r % mx2
    vc = jnp.stack([c0, c1, c2], axis=1)
    uh_s = jnp.where(row_valid, uh, _BIG)
    uh_pad = jnp.concatenate(
        [uh_s, jnp.full((_UH_PAD - N,), _BIG, jnp.int32)])
    offs = jnp.array(list(_prod((-1, 0, 1), repeat=3)), dtype=jnp.int32)
    nc = vc[None, :, :] + offs[:, None, :]
    valid = (jnp.all((nc >= 0) & (nc < mx[None, None, :]), axis=2)
             & row_valid[None, :])
    nh = nc[..., 0] * (mx1 * mx2) + nc[..., 1] * mx2 + nc[..., 2]
    gq = jnp.where(valid, nh, -2)
    gq2 = jnp.concatenate(
        [gq.reshape(-1),
         jnp.full((_NGROW * 128 - 27 * N,), -2, jnp.int32)])
    nbr = _sc_nbsearch(uh_pad, gq2)
    neigh = nbr[:27 * N].reshape(27, N)
    return inv, M, neigh


_NP = 53248
_RPW = _NP // 32
_CCH = 128
_NCH = _RPW // _CCH
_YTILE = 2048


def _conv_y(x, W):
    nt = _NP // _YTILE

    def body(x_ref, w_ref, y_ref):
        y_ref[...] = jnp.dot(x_ref[...], w_ref[0],
                             preferred_element_type=jnp.float32)

    return pl.pallas_call(
        body,
        grid=(nt, 27),
        in_specs=[
            pl.BlockSpec((_YTILE, _FD), lambda i, k: (i, 0)),
            pl.BlockSpec((1, _FD, _FD), lambda i, k: (k, 0, 0)),
        ],
        out_specs=pl.BlockSpec((_YTILE, _FD), lambda i, k: (k * nt + i, 0)),
        out_shape=jax.ShapeDtypeStruct((27 * _NP, _FD), jnp.float32),
        compiler_params=pltpu.CompilerParams(
            dimension_semantics=("arbitrary", "arbitrary")),
    )(x, W)


def _sc_gather_sum(Y, fidx):
    mesh = plsc.VectorSubcoreMesh(core_axis_name="c", subcore_axis_name="s",
                                  num_cores=2)

    @functools.partial(
        pl.kernel,
        out_type=jax.ShapeDtypeStruct((_NP, _FD), jnp.float32),
        mesh=mesh,
        compiler_params=pltpu.CompilerParams(needs_layout_passes=False),
        scratch_types=[
            pltpu.VMEM((27 * _RPW,), jnp.int32),
            pltpu.VMEM((_CCH, _FD), jnp.float32),
            pltpu.VMEM((_CCH, _FD), jnp.float32),
            pltpu.VMEM((_CCH, _FD), jnp.float32),
            pltpu.SemaphoreType.DMA,
            pltpu.SemaphoreType.DMA,
        ],
    )
    def k(y_ref, fidx_ref, out_ref, fx_v, acc_v, t0_v, t1_v, s0, s1):
        w = lax.axis_index("s") * 2 + lax.axis_index("c")
        r0 = w * _RPW

        @pl.loop(0, 27)
        def _(kk):
            pltpu.sync_copy(fidx_ref.at[pl.ds(kk * _NP + r0, _RPW)],
                            fx_v.at[pl.ds(kk * _RPW, _RPW)])

        @pl.loop(0, _NCH)
        def _(c):
            tb = (t0_v, t1_v)
            sb = (s0, s1)
            d = pltpu.async_copy(
                y_ref.at[fx_v.at[pl.ds(c * _CCH, _CCH)]], t0_v, s0)
            for kk in range(27):
                d.wait()
                if kk < 26:
                    d = pltpu.async_copy(
                        y_ref.at[fx_v.at[pl.ds((kk + 1) * _RPW + c * _CCH,
                                               _CCH)]],
                        tb[(kk + 1) % 2], sb[(kk + 1) % 2])
                t_v = tb[kk % 2]
                if kk == 0:
                    @pl.loop(0, _CCH)
                    def _(r):
                        for j in range(_FD // 16):
                            acc_v[r, pl.ds(j * 16, 16)] = \
                                t_v[r, pl.ds(j * 16, 16)]
                else:
                    @pl.loop(0, _CCH)
                    def _(r):
                        for j in range(_FD // 16):
                            plsc.addupdate(acc_v.at[r, pl.ds(j * 16, 16)],
                                           t_v[r, pl.ds(j * 16, 16)])
            pltpu.sync_copy(acc_v, out_ref.at[pl.ds(r0 + c * _CCH, _CCH), :])

    return k(Y, fidx)


def kernel(source_points, target_points, enc_W, enc_b, ln_g, ln_b, convW,
           bn_g, bn_b, h1_W, h1_b, h2_W, h2_b):
    sc = source_points - source_points.mean(axis=0, keepdims=True)
    tc = target_points - target_points.mean(axis=0, keepdims=True)
    s_inv, s_M, s_nb = _structure(sc)
    t_inv, t_M, t_nb = _structure(tc)
    N = source_points.shape[0]

    def encode(p):
        x = p @ enc_W + enc_b
        m = x.mean(axis=-1, keepdims=True)
        v = ((x - m) ** 2).mean(axis=-1, keepdims=True)
        x = (x - m) / jnp.sqrt(v + 1e-5) * ln_g + ln_b
        return jax.nn.relu(x)

    def vox_mean(feats, inv):
        s = jax.ops.segment_sum(feats, inv, num_segments=_NP)
        c = jax.ops.segment_sum(jnp.ones((feats.shape[0],), feats.dtype),
                                inv, num_segments=_NP)
        return s / jnp.where(c > 0, c, jnp.ones_like(c))[:, None]

    def mk_fidx(nb):
        cols = jnp.concatenate(
            [nb, jnp.full((27, _NP - N), -1, jnp.int32)], axis=1)
        return (jnp.where(cols >= 0, cols, _NP - 1)
                + jnp.arange(27, dtype=jnp.int32)[:, None] * _NP).reshape(-1)

    def bn(x, g, b, mask, Mf):
        m = jnp.where(mask[:, None], x, 0.0).sum(axis=0) / Mf
        v = jnp.where(mask[:, None], (x - m) ** 2, 0.0).sum(axis=0) / Mf
        return (x - m) / jnp.sqrt(v + 1e-5) * g + b

    def blocks(f, fidx, mask, Mf):
        x = f
        for bi in range(_NBLK):
            idn = x
            y = _sc_gather_sum(_conv_y(x, convW[bi, 0]), fidx)
            y = jax.nn.relu(bn(y, bn_g[bi, 0], bn_b[bi, 0], mask, Mf))
            y = jnp.where(mask[:, None], y, 0.0)
            z = _sc_gather_sum(_conv_y(y, convW[bi, 1]), fidx)
            z = bn(z, bn_g[bi, 1], bn_b[bi, 1], mask, Mf) + idn
            x = jnp.where(mask[:, None], jax.nn.relu(z), 0.0)
        return x

    rows = jnp.arange(_NP, dtype=jnp.int32)
    s_mask = rows < s_M
    t_mask = rows < t_M
    s_Mf = s_M.astype(jnp.float32)
    t_Mf = t_M.astype(jnp.float32)

    s_feats = blocks(vox_mean(encode(source_points), s_inv),
                     mk_fidx(s_nb), s_mask, s_Mf)
    t_feats = blocks(vox_mean(encode(target_points), t_inv),
                     mk_fidx(t_nb), t_mask, t_Mf)
    sg = jnp.where(s_mask[:, None], s_feats, -jnp.inf).max(axis=0)
    tg = jnp.where(t_mask[:, None], t_feats, -jnp.inf).max(axis=0)
    comb = sg + tg
    h = jax.nn.relu(comb @ h1_W + h1_b)
    return h @ h2_W + h2_b
